# baseline (speedup 1.0000x reference)
import functools

import jax
import jax.numpy as jnp
from jax import lax
from jax.experimental import pallas as pl
from jax.experimental.pallas import tpu as pltpu
from jax.experimental.pallas import tpu_sc as plsc

D_MODEL = 64
SCALE = 8.0
NUM_CORES = 2
NUM_SUBCORES = 16
NUM_WORKERS = NUM_CORES * NUM_SUBCORES
CHUNK = 128
DCHUNK = 320
ROWS_PER_ITER = 4
LANES = 16


@functools.lru_cache(maxsize=None)
def _make_depad(V: int):
    n_chunks = V // DCHUNK
    mesh = plsc.VectorSubcoreMesh(core_axis_name="c", subcore_axis_name="s")

    @functools.partial(
        pl.kernel,
        mesh=mesh,
        out_type=jax.ShapeDtypeStruct((V // 2, 2 * D_MODEL), jnp.float32),
        scratch_types=[
            pltpu.VMEM((DCHUNK, D_MODEL), jnp.float32),
            pltpu.VMEM((DCHUNK, D_MODEL), jnp.float32),
            pltpu.VMEM((DCHUNK // 2, 2 * D_MODEL), jnp.float32),
            pltpu.VMEM((DCHUNK // 2, 2 * D_MODEL), jnp.float32),
            pltpu.SemaphoreType.DMA,
            pltpu.SemaphoreType.DMA,
        ],
    )
    def depad(t_hbm, o_hbm, b0, b1, p0, p1, sem0, sem1):
        wid = lax.axis_index("s") * NUM_CORES + lax.axis_index("c")
        bufs = (b0, b1)
        pks = (p0, p1)
        sems = (sem0, sem1)
        my_chunks = -(-n_chunks // NUM_WORKERS)

        def chunk_id(g):
            return jnp.minimum(g * NUM_WORKERS + wid, n_chunks - 1)

        pltpu.async_copy(
            t_hbm.at[pl.ds(pl.multiple_of(chunk_id(0) * DCHUNK, 8), DCHUNK)],
            b0,
            sem0,
        )

        def super_body(h, carry):
            for b in range(2):
                g = 2 * h + b
                ch = chunk_id(g)

                @pl.when(g + 1 < my_chunks)
                def _():
                    nch = chunk_id(g + 1)
                    pltpu.async_copy(
                        t_hbm.at[pl.ds(pl.multiple_of(nch * DCHUNK, 8), DCHUNK)],
                        bufs[1 - b],
                        sems[1 - b],
                    )

                pltpu.make_async_copy(
                    t_hbm.at[pl.ds(pl.multiple_of(ch * DCHUNK, 8), DCHUNK)],
                    bufs[b],
                    sems[b],
                ).wait()

                def pack_body(i, carry2):
                    for u in range(ROWS_PER_ITER):
                        m = i * ROWS_PER_ITER + u
                        for j in range(D_MODEL // LANES):
                            sl = pl.ds(j * LANES, LANES)
                            pks[b][m, sl] = bufs[b][2 * m, sl]
                            sl2 = pl.ds(D_MODEL + j * LANES, LANES)
                            pks[b][m, sl2] = bufs[b][2 * m + 1, sl]
                    return carry2

                lax.fori_loop(0, DCHUNK // (2 * ROWS_PER_ITER), pack_body, 0)
                oo = pl.multiple_of(ch * (DCHUNK // 2), 8)
                pltpu.sync_copy(pks[b], o_hbm.at[pl.ds(oo, DCHUNK // 2)])
            return carry

        lax.fori_loop(0, my_chunks // 2, super_body, 0)

    return depad


@functools.lru_cache(maxsize=None)
def _make_gather(B: int, V: int):
    b_per_w = B // NUM_WORKERS
    n_chunks = b_per_w // CHUNK
    mesh = plsc.VectorSubcoreMesh(core_axis_name="c", subcore_axis_name="s")

    @functools.partial(
        pl.kernel,
        mesh=mesh,
        out_type=jax.ShapeDtypeStruct((B // 2, 2 * D_MODEL), jnp.float32),
        scratch_types=[
            pltpu.VMEM((b_per_w,), jnp.int32),
            pltpu.VMEM((CHUNK, D_MODEL), jnp.float32),
            pltpu.VMEM((CHUNK, D_MODEL), jnp.float32),
            pltpu.VMEM((CHUNK // 2, 2 * D_MODEL), jnp.float32),
            pltpu.VMEM((CHUNK // 2, 2 * D_MODEL), jnp.float32),
            pltpu.SemaphoreType.DMA,
            pltpu.SemaphoreType.DMA,
        ],
        compiler_params=pltpu.CompilerParams(use_tc_tiling_on_sc=False),
    )
    def emb(x_hbm, t_hbm, out_hbm, idx_all, rows0, rows1, o20, o21, sem0, sem1):
        wid = lax.axis_index("s") * NUM_CORES + lax.axis_index("c")
        base = pl.multiple_of(wid * b_per_w, 8)
        obase = pl.multiple_of(wid * (b_per_w // 2), 8)
        rows = (rows0, rows1)
        o2s = (o20, o21)
        sems = (sem0, sem1)

        pltpu.sync_copy(x_hbm.at[pl.ds(base, b_per_w)], idx_all)
        pltpu.async_copy(t_hbm.at[idx_all.at[pl.ds(0, CHUNK)]], rows0, sem0)

        def super_body(h, carry):
            for b in range(2):
                g = 2 * h + b

                @pl.when(g + 1 < n_chunks)
                def _():
                    nxt = pl.multiple_of((g + 1) * CHUNK, 8)
                    pltpu.async_copy(
                        t_hbm.at[idx_all.at[pl.ds(nxt, CHUNK)]],
                        rows[1 - b],
                        sems[1 - b],
                    )

                goff = pl.multiple_of(g * CHUNK, 8)
                pltpu.make_async_copy(
                    t_hbm.at[idx_all.at[pl.ds(goff, CHUNK)]], rows[b], sems[b]
                ).wait()

                def pack_body(i, carry2):
                    for u in range(ROWS_PER_ITER):
                        m = i * ROWS_PER_ITER + u
                        for j in range(D_MODEL // LANES):
                            sl = pl.ds(j * LANES, LANES)
                            o2s[b][m, sl] = rows[b][2 * m, sl] * SCALE
                            sl2 = pl.ds(D_MODEL + j * LANES, LANES)
                            o2s[b][m, sl2] = rows[b][2 * m + 1, sl] * SCALE
                    return carry2

                lax.fori_loop(0, CHUNK // (2 * ROWS_PER_ITER), pack_body, 0)
                oo = pl.multiple_of(obase + g * (CHUNK // 2), 8)
                pltpu.sync_copy(o2s[b], out_hbm.at[pl.ds(oo, CHUNK // 2)])
            return carry

        lax.fori_loop(0, n_chunks // 2, super_body, 0)

    return emb


def kernel(x, table):
    B = x.size
    V = table.shape[0]
    t_pairs = _make_depad(V)(table)
    t_lin = t_pairs.reshape(V, D_MODEL)
    o2 = _make_gather(B, V)(x.reshape(-1), t_lin)
    return o2.reshape(*x.shape, D_MODEL)


# speedup vs baseline: 1.2370x; 1.2370x over previous
import functools

import jax
import jax.numpy as jnp
from jax import lax
from jax.experimental import pallas as pl
from jax.experimental.pallas import tpu as pltpu
from jax.experimental.pallas import tpu_sc as plsc

D_MODEL = 64
SCALE = 8.0
NUM_CORES = 2
NUM_SUBCORES = 16
NUM_WORKERS = NUM_CORES * NUM_SUBCORES
CHUNK = 256
ROWS_PER_ITER = 4
LANES = 16


@functools.lru_cache(maxsize=None)
def _make_gather(B: int, V: int):
    b_per_w = B // NUM_WORKERS
    n_chunks = b_per_w // CHUNK
    mesh = plsc.VectorSubcoreMesh(core_axis_name="c", subcore_axis_name="s")

    @functools.partial(
        pl.kernel,
        mesh=mesh,
        out_type=jax.ShapeDtypeStruct((B // 2, 2 * D_MODEL), jnp.float32),
        scratch_types=[
            pltpu.VMEM((b_per_w,), jnp.int32),
            pltpu.VMEM((CHUNK, D_MODEL), jnp.float32),
            pltpu.VMEM((CHUNK, D_MODEL), jnp.float32),
            pltpu.VMEM((CHUNK // 2, 2 * D_MODEL), jnp.float32),
            pltpu.VMEM((CHUNK // 2, 2 * D_MODEL), jnp.float32),
            pltpu.SemaphoreType.DMA,
            pltpu.SemaphoreType.DMA,
        ],
        compiler_params=pltpu.CompilerParams(use_tc_tiling_on_sc=False),
    )
    def emb(x_hbm, t_hbm, out_hbm, idx_all, rows0, rows1, o20, o21, sem0, sem1):
        wid = lax.axis_index("s") * NUM_CORES + lax.axis_index("c")
        base = pl.multiple_of(wid * b_per_w, 8)
        obase = pl.multiple_of(wid * (b_per_w // 2), 8)
        rows = (rows0, rows1)
        o2s = (o20, o21)
        sems = (sem0, sem1)

        pltpu.sync_copy(x_hbm.at[pl.ds(base, b_per_w)], idx_all)
        pltpu.async_copy(t_hbm.at[idx_all.at[pl.ds(0, CHUNK)]], rows0, sem0)

        def super_body(h, carry):
            for b in range(2):
                g = 2 * h + b

                @pl.when(g + 1 < n_chunks)
                def _():
                    nxt = pl.multiple_of((g + 1) * CHUNK, 8)
                    pltpu.async_copy(
                        t_hbm.at[idx_all.at[pl.ds(nxt, CHUNK)]],
                        rows[1 - b],
                        sems[1 - b],
                    )

                goff = pl.multiple_of(g * CHUNK, 8)
                pltpu.make_async_copy(
                    t_hbm.at[idx_all.at[pl.ds(goff, CHUNK)]], rows[b], sems[b]
                ).wait()

                def pack_body(i, carry2):
                    for u in range(ROWS_PER_ITER):
                        m = i * ROWS_PER_ITER + u
                        for j in range(D_MODEL // LANES):
                            sl = pl.ds(j * LANES, LANES)
                            o2s[b][m, sl] = rows[b][2 * m, sl] * SCALE
                            sl2 = pl.ds(D_MODEL + j * LANES, LANES)
                            o2s[b][m, sl2] = rows[b][2 * m + 1, sl] * SCALE
                    return carry2

                lax.fori_loop(0, CHUNK // (2 * ROWS_PER_ITER), pack_body, 0)
                oo = pl.multiple_of(obase + g * (CHUNK // 2), 8)
                pltpu.sync_copy(o2s[b], out_hbm.at[pl.ds(oo, CHUNK // 2)])
            return carry

        lax.fori_loop(0, n_chunks // 2, super_body, 0)

    return emb


def kernel(x, table):
    B = x.size
    V = table.shape[0]
    t_feat = lax.optimization_barrier(table.T)
    t_lin = t_feat.T
    o2 = _make_gather(B, V)(x.reshape(-1), t_lin)
    return o2.reshape(*x.shape, D_MODEL)
